# baseline (speedup 1.0000x reference)
import jax
import jax.numpy as jnp
from jax.experimental import pallas as pl
from jax.experimental.pallas import tpu as pltpu

EMBED_DIM = 64
FFN_DIM = 128
NUM_EXPERTS = 4


def _moe_half(xh, wg, w1s, w2s, b1s, b2s):
    D, F, E = EMBED_DIM, FFN_DIM, NUM_EXPERTS
    lT = jax.lax.dot_general(
        wg, xh, (((0,), (1,)), ((), ())),
        preferred_element_type=jnp.float32)

    e_iota = jax.lax.broadcasted_iota(jnp.int32, lT.shape, 0)
    m1 = jnp.max(lT, axis=0, keepdims=True)
    idx1 = jnp.min(jnp.where(lT == m1, e_iota, E), axis=0, keepdims=True)
    masked = jnp.where(e_iota == idx1, -jnp.inf, lT)
    m2 = jnp.max(masked, axis=0, keepdims=True)
    idx2 = jnp.min(jnp.where(masked == m2, e_iota, E), axis=0, keepdims=True)
    p1 = 1.0 / (1.0 + jnp.exp(m2 - m1))
    p2 = 1.0 - p1
    wT = (jnp.where(e_iota == idx1, p1, 0.0)
          + jnp.where(e_iota == idx2, p2, 0.0))

    h = jax.lax.dot_general(
        xh, w1s, (((1,), (0,)), ((), ())),
        preferred_element_type=jnp.float32) + b1s
    h = jnp.maximum(h, 0.0)

    out_all = jax.lax.dot_general(
        h, w2s, (((1,), (0,)), ((), ())),
        preferred_element_type=jnp.float32)

    lane_e = jax.lax.broadcasted_iota(jnp.int32, (E, E * D), 1) // D
    sub_e = jax.lax.broadcasted_iota(jnp.int32, (E, E * D), 0)
    ex = jnp.where(lane_e == sub_e, 1.0, 0.0).astype(jnp.float32)
    wcol = jax.lax.dot_general(
        wT, ex, (((0,), (0,)), ((), ())),
        preferred_element_type=jnp.float32)
    scaled = (out_all + b2s) * wcol

    return (scaled[:, 0:D] + scaled[:, D:2 * D]
            + scaled[:, 2 * D:3 * D] + scaled[:, 3 * D:4 * D])


def _moe_kernel(x_ref, wg_ref, w1_ref, b1_ref, w2_ref, b2_ref, o_ref,
                w1s, w2s, b1s, b2s):
    D, F, E = EMBED_DIM, FFN_DIM, NUM_EXPERTS

    @pl.when(pl.program_id(0) == 0)
    def _prep():
        w2s[:] = jnp.zeros((E * F, E * D), jnp.float32)
        for e in range(E):
            w1s[:, e * F:(e + 1) * F] = w1_ref[e]
            w2s[e * F:(e + 1) * F, e * D:(e + 1) * D] = w2_ref[e]
            b1s[0:1, e * F:(e + 1) * F] = b1_ref[e:e + 1, :]
            b2s[0:1, e * D:(e + 1) * D] = b2_ref[e:e + 1, :]

    xp = x_ref[:]
    wg, w1, w2, b1, b2 = wg_ref[:], w1s[:], w2s[:], b1s[:], b2s[:]
    o_ref[:, 0:D] = _moe_half(xp[:, 0:D], wg, w1, w2, b1, b2)
    o_ref[:, D:2 * D] = _moe_half(xp[:, D:2 * D], wg, w1, w2, b1, b2)


def kernel(x, Wg, W1, b1, W2, b2):
    x = x.reshape(-1, x.shape[-1])
    T, D = x.shape
    E, _, F = W1.shape
    T2 = T // 2
    xp = x.reshape(T2, 2 * D)

    TB2 = 512
    grid = (T2 // TB2,)
    out = pl.pallas_call(
        _moe_kernel,
        grid=grid,
        in_specs=[
            pl.BlockSpec((TB2, 2 * D), lambda i: (i, 0)),
            pl.BlockSpec((D, E), lambda i: (0, 0)),
            pl.BlockSpec((E, D, F), lambda i: (0, 0, 0)),
            pl.BlockSpec((E, F), lambda i: (0, 0)),
            pl.BlockSpec((E, F, D), lambda i: (0, 0, 0)),
            pl.BlockSpec((E, D), lambda i: (0, 0)),
        ],
        out_specs=pl.BlockSpec((TB2, 2 * D), lambda i: (i, 0)),
        out_shape=jax.ShapeDtypeStruct((T2, 2 * D), jnp.float32),
        scratch_shapes=[
            pltpu.VMEM((D, E * F), jnp.float32),
            pltpu.VMEM((E * F, E * D), jnp.float32),
            pltpu.VMEM((1, E * F), jnp.float32),
            pltpu.VMEM((1, E * D), jnp.float32),
        ],
        compiler_params=pltpu.CompilerParams(
            dimension_semantics=("arbitrary",)),
    )(xp, Wg, W1, b1, W2, b2)
    return out.reshape(T, D)


# speedup vs baseline: 3.0137x; 3.0137x over previous
import jax
import jax.numpy as jnp
from jax.experimental import pallas as pl
from jax.experimental.pallas import tpu as pltpu

EMBED_DIM = 64
FFN_DIM = 128
NUM_EXPERTS = 4


def _moe_kernel(x_ref, wg_ref, w1_ref, b1_ref, w2t_ref, b2_ref, o_ref,
                wgs, w1s, w2s, b1s, b2s):
    D, F, E = EMBED_DIM, FFN_DIM, NUM_EXPERTS

    @pl.when(pl.program_id(0) == 0)
    def _prep():
        wgs[:] = jnp.transpose(wg_ref[:], (1, 0))
        w2s[:] = jnp.zeros((E * D, E * F), jnp.float32)
        for e in range(E):
            w1s[e * F:(e + 1) * F, :] = jnp.transpose(w1_ref[e], (1, 0))
            w2s[e * D:(e + 1) * D, e * F:(e + 1) * F] = w2t_ref[e]
            b1s[e * F:(e + 1) * F, 0:1] = jnp.transpose(b1_ref[e:e + 1, :],
                                                        (1, 0))
            b2s[e * D:(e + 1) * D, 0:1] = jnp.transpose(b2_ref[e:e + 1, :],
                                                        (1, 0))

    xb = x_ref[:]
    lT = jax.lax.dot_general(
        wgs[:], xb, (((1,), (0,)), ((), ())),
        preferred_element_type=jnp.float32)

    e_iota = jax.lax.broadcasted_iota(jnp.int32, lT.shape, 0)
    m1 = jnp.max(lT, axis=0, keepdims=True)
    idx1 = jnp.min(jnp.where(lT == m1, e_iota, E), axis=0, keepdims=True)
    masked = jnp.where(e_iota == idx1, -jnp.inf, lT)
    m2 = jnp.max(masked, axis=0, keepdims=True)
    idx2 = jnp.min(jnp.where(masked == m2, e_iota, E), axis=0, keepdims=True)
    p1 = 1.0 / (1.0 + jnp.exp(m2 - m1))
    p2 = 1.0 - p1
    wT = (jnp.where(e_iota == idx1, p1, 0.0)
          + jnp.where(e_iota == idx2, p2, 0.0))

    hT = jax.lax.dot_general(
        w1s[:], xb, (((1,), (0,)), ((), ())),
        preferred_element_type=jnp.float32) + b1s[:]
    hT = jnp.maximum(hT, 0.0)

    out_aT = jax.lax.dot_general(
        w2s[:], hT, (((1,), (0,)), ((), ())),
        preferred_element_type=jnp.float32)

    row_e = jax.lax.broadcasted_iota(jnp.int32, (E * D, E), 0) // D
    col_e = jax.lax.broadcasted_iota(jnp.int32, (E * D, E), 1)
    ex = jnp.where(row_e == col_e, 1.0, 0.0).astype(jnp.float32)
    wcolT = jax.lax.dot_general(
        ex, wT, (((1,), (0,)), ((), ())),
        preferred_element_type=jnp.float32)
    scaled = (out_aT + b2s[:]) * wcolT

    o_ref[:] = (scaled[0:D, :] + scaled[D:2 * D, :]
                + scaled[2 * D:3 * D, :] + scaled[3 * D:4 * D, :])


def kernel(x, Wg, W1, b1, W2, b2):
    x = x.reshape(-1, x.shape[-1])
    T, D = x.shape
    E, _, F = W1.shape
    xT = x.T
    W2t = W2.transpose(0, 2, 1)

    TB = 1024
    grid = (T // TB,)
    outT = pl.pallas_call(
        _moe_kernel,
        grid=grid,
        in_specs=[
            pl.BlockSpec((D, TB), lambda i: (0, i)),
            pl.BlockSpec((D, E), lambda i: (0, 0)),
            pl.BlockSpec((E, D, F), lambda i: (0, 0, 0)),
            pl.BlockSpec((E, F), lambda i: (0, 0)),
            pl.BlockSpec((E, D, F), lambda i: (0, 0, 0)),
            pl.BlockSpec((E, D), lambda i: (0, 0)),
        ],
        out_specs=pl.BlockSpec((D, TB), lambda i: (0, i)),
        out_shape=jax.ShapeDtypeStruct((D, T), jnp.float32),
        scratch_shapes=[
            pltpu.VMEM((E, D), jnp.float32),
            pltpu.VMEM((E * F, D), jnp.float32),
            pltpu.VMEM((E * D, E * F), jnp.float32),
            pltpu.VMEM((E * F, 1), jnp.float32),
            pltpu.VMEM((E * D, 1), jnp.float32),
        ],
        compiler_params=pltpu.CompilerParams(
            dimension_semantics=("arbitrary",)),
    )(xT, Wg, W1, b1, W2t, b2)
    return outT.T
